# baseline (speedup 1.0000x reference)
import functools
import jax
import jax.numpy as jnp
from jax import lax
from jax.experimental import pallas as pl
from jax.experimental.pallas import tpu as pltpu
from jax.experimental.pallas import tpu_sc as plsc

B, L = 8, 448
SEQ_EMB = 32
RES_DIM = 128
PAIR_DIM = 64
MAX_REL = 32
NREL = 2 * MAX_REL + 1
NUM_EMB = 5

_HI = jax.lax.Precision.HIGHEST

NC, NS = 2, 16
NW = NC * NS
ROWS = B * L
RPW = ROWS // NW
JU = 4


def _prep_body(seq_ref, dih_ref, ent_ref, acc_ref, con_ref, emb_ref, pe_ref,
               rel_emb_ref, Wr_ref, br_ref, Wp_ref, bp_ref,
               res_out, relproj_out):
    seq = seq_ref[...]
    onehot = (seq[..., None] ==
              jax.lax.broadcasted_iota(jnp.int32, (B, L, NUM_EMB), 2)
              ).astype(jnp.float32)
    M = jax.lax.dot_general(emb_ref[...], Wr_ref[0:SEQ_EMB, :],
                            (((1,), (0,)), ((), ())), precision=_HI)
    res = jax.lax.dot_general(onehot.reshape(B * L, NUM_EMB), M,
                              (((1,), (0,)), ((), ())), precision=_HI)
    res = res + jax.lax.dot_general(
        dih_ref[...].reshape(B * L, 4), Wr_ref[SEQ_EMB:SEQ_EMB + 4, :],
        (((1,), (0,)), ((), ())), precision=_HI)
    res = res.reshape(B, L, RES_DIM)
    res = res + ent_ref[...][..., None] * Wr_ref[SEQ_EMB + 4, :][None, None, :]
    res = res + acc_ref[...][..., None] * Wr_ref[SEQ_EMB + 5, :][None, None, :]
    res = res + con_ref[...][..., None] * Wr_ref[SEQ_EMB + 6, :][None, None, :]
    res = res + br_ref[...][None, None, :]
    res = res + pe_ref[0, :L, :][None]
    res_out[...] = res
    relproj_out[...] = jax.lax.dot_general(
        rel_emb_ref[...], Wp_ref[2:, :], (((1,), (0,)), ((), ())),
        precision=_HI) + bp_ref[...][None, :]


def _sc_pair_body(relproj_hbm, w01_hbm, p_hbm, c_hbm, out_hbm,
                  relv, wv, pv, cv, ov,
                  psem0, psem1, csem0, csem1, osem0, osem1):
    wid = lax.axis_index("s") * NC + lax.axis_index("c")
    base_row = wid * RPW
    pltpu.sync_copy(relproj_hbm, relv)
    pltpu.sync_copy(w01_hbm, wv)
    w0 = [wv[0, pl.ds(d * 16, 16)] for d in range(4)]
    w1 = [wv[1, pl.ds(d * 16, 16)] for d in range(4)]

    def row_bi(r):
        row = base_row + r
        b = row // L
        return b, row - b * L

    for s, (ps, cs) in ((0, (psem0, csem0)), (1, (psem1, csem1))):
        b, i = row_bi(s)
        pltpu.make_async_copy(p_hbm.at[b, i], pv.at[s], ps).start()
        pltpu.make_async_copy(c_hbm.at[b, i], cv.at[s], cs).start()

    def group_body(g, _):
        for s, (ps, cs, osm) in ((0, (psem0, csem0, osem0)),
                                 (1, (psem1, csem1, osem1))):
            r = g * 2 + s
            b, i = row_bi(r)
            pltpu.make_async_copy(p_hbm.at[b, i], pv.at[s], ps).wait()
            pltpu.make_async_copy(c_hbm.at[b, i], cv.at[s], cs).wait()
            @pl.when(g > 0)
            def _():
                pb_, pi_ = row_bi(r - 2)
                pltpu.make_async_copy(ov.at[pl.ds(s * L, L)],
                                      out_hbm.at[pb_, pi_], osm).wait()

            def j_body(jg, _):
                chp = pv[s, pl.ds(jg * 16, 16)]
                chc = cv[s, pl.ds(jg * 16, 16)]
                jbase = jg * 16
                for u in range(16):
                    j = jbase + u
                    pb = jnp.full((16,), chp[u], jnp.float32)
                    cb = jnp.full((16,), chc[u], jnp.float32)
                    k = jnp.clip(j - i + MAX_REL, 0, 2 * MAX_REL)
                    for d in range(4):
                        rel = relv[k, pl.ds(d * 16, 16)]
                        ov[s * L + j, pl.ds(d * 16, 16)] = (
                            pb * w0[d] + cb * w1[d] + rel)
                return 0

            lax.fori_loop(0, L // 16, j_body, 0, unroll=False)
            pltpu.make_async_copy(ov.at[pl.ds(s * L, L)],
                                  out_hbm.at[b, i], osm).start()

            @pl.when(r + 2 < RPW)
            def _():
                nb, ni = row_bi(r + 2)
                pltpu.make_async_copy(p_hbm.at[nb, ni], pv.at[s], ps).start()
                pltpu.make_async_copy(c_hbm.at[nb, ni], cv.at[s], cs).start()
        return 0

    lax.fori_loop(0, RPW // 2, group_body, 0, unroll=False)
    for s, osm in ((0, osem0), (1, osem1)):
        b, i = row_bi(RPW - 2 + s)
        pltpu.make_async_copy(ov.at[pl.ds(s * L, L)],
                              out_hbm.at[b, i], osm).wait()


@functools.lru_cache(maxsize=1)
def _sc_pair():
  return pl.kernel(
    _sc_pair_body,
    out_type=jax.ShapeDtypeStruct((B, L, L, PAIR_DIM), jnp.float32),
    mesh=plsc.VectorSubcoreMesh(core_axis_name="c", subcore_axis_name="s",
                                num_cores=NC, num_subcores=NS),
    scratch_types=[
        pltpu.VMEM((NREL, PAIR_DIM), jnp.float32),
        pltpu.VMEM((2, PAIR_DIM), jnp.float32),
        pltpu.VMEM((2, L), jnp.float32),
        pltpu.VMEM((2, L), jnp.float32),
        pltpu.VMEM((2 * L, PAIR_DIM), jnp.float32),
        pltpu.SemaphoreType.DMA,
        pltpu.SemaphoreType.DMA,
        pltpu.SemaphoreType.DMA,
        pltpu.SemaphoreType.DMA,
        pltpu.SemaphoreType.DMA,
        pltpu.SemaphoreType.DMA,
    ],
  )


@jax.jit
def _impl(sequence_int, dihedral_features, pairing_probs, positional_entropy,
          coupling_matrix, accessibility, conservation, emb_table, pe,
          rel_emb, W_res, b_res, W_pair, b_pair):
    res, relproj = pl.pallas_call(
        _prep_body,
        out_shape=(
            jax.ShapeDtypeStruct((B, L, RES_DIM), jnp.float32),
            jax.ShapeDtypeStruct((NREL, PAIR_DIM), jnp.float32),
        ),
    )(sequence_int.astype(jnp.int32), dihedral_features, positional_entropy,
      accessibility, conservation, emb_table, pe, rel_emb, W_res, b_res,
      W_pair, b_pair)

    pair = _sc_pair()(relproj, W_pair[0:2, :], pairing_probs, coupling_matrix)
    return res, pair


def kernel(sequence_int, mask, dihedral_features, pairing_probs,
           positional_entropy, coupling_matrix, accessibility, conservation,
           emb_table, pe, rel_emb, W_res, b_res, W_pair, b_pair):
    res, pair = _impl(sequence_int, dihedral_features, pairing_probs,
                      positional_entropy, coupling_matrix, accessibility,
                      conservation, emb_table, pe, rel_emb, W_res, b_res,
                      W_pair, b_pair)
    return res, pair, mask


# speedup vs baseline: 1.5959x; 1.5959x over previous
import functools
import jax
import jax.numpy as jnp
from jax import lax
from jax.experimental import pallas as pl
from jax.experimental.pallas import tpu as pltpu
from jax.experimental.pallas import tpu_sc as plsc

B, L = 8, 448
SEQ_EMB = 32
RES_DIM = 128
PAIR_DIM = 64
MAX_REL = 32
NREL = 2 * MAX_REL + 1
NUM_EMB = 5

_HI = jax.lax.Precision.HIGHEST

NC, NS = 2, 16
NW = NC * NS
ROWS = B * L
RPW = ROWS // NW
JU = 4


def _prep_body(seq_ref, dih_ref, ent_ref, acc_ref, con_ref, emb_ref, pe_ref,
               rel_emb_ref, Wr_ref, br_ref, Wp_ref, bp_ref,
               res_out, relproj_out):
    seq = seq_ref[...]
    onehot = (seq[..., None] ==
              jax.lax.broadcasted_iota(jnp.int32, (B, L, NUM_EMB), 2)
              ).astype(jnp.float32)
    M = jax.lax.dot_general(emb_ref[...], Wr_ref[0:SEQ_EMB, :],
                            (((1,), (0,)), ((), ())), precision=_HI)
    res = jax.lax.dot_general(onehot.reshape(B * L, NUM_EMB), M,
                              (((1,), (0,)), ((), ())), precision=_HI)
    res = res + jax.lax.dot_general(
        dih_ref[...].reshape(B * L, 4), Wr_ref[SEQ_EMB:SEQ_EMB + 4, :],
        (((1,), (0,)), ((), ())), precision=_HI)
    res = res.reshape(B, L, RES_DIM)
    res = res + ent_ref[...][..., None] * Wr_ref[SEQ_EMB + 4, :][None, None, :]
    res = res + acc_ref[...][..., None] * Wr_ref[SEQ_EMB + 5, :][None, None, :]
    res = res + con_ref[...][..., None] * Wr_ref[SEQ_EMB + 6, :][None, None, :]
    res = res + br_ref[...][None, None, :]
    res = res + pe_ref[0, :L, :][None]
    res_out[...] = res
    relproj_out[...] = jax.lax.dot_general(
        rel_emb_ref[...], Wp_ref[2:, :], (((1,), (0,)), ((), ())),
        precision=_HI) + bp_ref[...][None, :]


_GDN = lax.GatherDimensionNumbers(offset_dims=(), collapsed_slice_dims=(0,),
                                  start_index_map=(0,))


def _bcast(ch, u):
    return lax.gather(ch, jnp.full((16, 1), u, jnp.int32), _GDN, (1,),
                      mode=lax.GatherScatterMode.PROMISE_IN_BOUNDS)


def _sc_pair_body(relproj_hbm, w01_hbm, p_hbm, c_hbm, out_hbm,
                  relv, wv, pv, cv, ov,
                  psem0, psem1, csem0, csem1, osem0, osem1):
    wid = lax.axis_index("s") * NC + lax.axis_index("c")
    base_row = wid * RPW
    pltpu.sync_copy(relproj_hbm, relv)
    pltpu.sync_copy(w01_hbm, wv)
    w0 = [wv[0, pl.ds(d * 16, 16)] for d in range(4)]
    w1 = [wv[1, pl.ds(d * 16, 16)] for d in range(4)]

    def row_bi(r):
        row = base_row + r
        b = row // L
        return b, row - b * L

    for s, (ps, cs) in ((0, (psem0, csem0)), (1, (psem1, csem1))):
        b, i = row_bi(s)
        pltpu.make_async_copy(p_hbm.at[b, i], pv.at[s], ps).start()
        pltpu.make_async_copy(c_hbm.at[b, i], cv.at[s], cs).start()

    def group_body(g, _):
        for s, (ps, cs, osm) in ((0, (psem0, csem0, osem0)),
                                 (1, (psem1, csem1, osem1))):
            r = g * 2 + s
            b, i = row_bi(r)
            pltpu.make_async_copy(p_hbm.at[b, i], pv.at[s], ps).wait()
            pltpu.make_async_copy(c_hbm.at[b, i], cv.at[s], cs).wait()
            @pl.when(g > 0)
            def _():
                pb_, pi_ = row_bi(r - 2)
                pltpu.make_async_copy(ov.at[pl.ds(s * L, L)],
                                      out_hbm.at[pb_, pi_], osm).wait()

            def j_body(jg, _):
                jbase = jg * 16
                chp = pv[s, pl.ds(jbase, 16)]
                chc = cv[s, pl.ds(jbase, 16)]
                klo = jnp.clip(jbase - i + MAX_REL, 0, 2 * MAX_REL)
                khi = jnp.clip(jbase + 15 - i + MAX_REL, 0, 2 * MAX_REL)

                @pl.when(klo == khi)
                def _():
                    rels = [relv[klo, pl.ds(d * 16, 16)] for d in range(4)]
                    for u in range(16):
                        pb = _bcast(chp, u)
                        cb = _bcast(chc, u)
                        row = s * L + jbase + u
                        for d in range(4):
                            ov[row, pl.ds(d * 16, 16)] = (
                                pb * w0[d] + cb * w1[d] + rels[d])

                @pl.when(klo != khi)
                def _():
                    for u in range(16):
                        pb = _bcast(chp, u)
                        cb = _bcast(chc, u)
                        k = jnp.clip(jbase + u - i + MAX_REL, 0, 2 * MAX_REL)
                        row = s * L + jbase + u
                        for d in range(4):
                            ov[row, pl.ds(d * 16, 16)] = (
                                pb * w0[d] + cb * w1[d]
                                + relv[k, pl.ds(d * 16, 16)])
                return 0

            lax.fori_loop(0, L // 16, j_body, 0, unroll=False)
            pltpu.make_async_copy(ov.at[pl.ds(s * L, L)],
                                  out_hbm.at[b, i], osm).start()

            @pl.when(r + 2 < RPW)
            def _():
                nb, ni = row_bi(r + 2)
                pltpu.make_async_copy(p_hbm.at[nb, ni], pv.at[s], ps).start()
                pltpu.make_async_copy(c_hbm.at[nb, ni], cv.at[s], cs).start()
        return 0

    lax.fori_loop(0, RPW // 2, group_body, 0, unroll=False)
    for s, osm in ((0, osem0), (1, osem1)):
        b, i = row_bi(RPW - 2 + s)
        pltpu.make_async_copy(ov.at[pl.ds(s * L, L)],
                              out_hbm.at[b, i], osm).wait()


@functools.lru_cache(maxsize=1)
def _sc_pair():
  return pl.kernel(
    _sc_pair_body,
    out_type=jax.ShapeDtypeStruct((B, L, L, PAIR_DIM), jnp.float32),
    mesh=plsc.VectorSubcoreMesh(core_axis_name="c", subcore_axis_name="s",
                                num_cores=NC, num_subcores=NS),
    scratch_types=[
        pltpu.VMEM((NREL, PAIR_DIM), jnp.float32),
        pltpu.VMEM((2, PAIR_DIM), jnp.float32),
        pltpu.VMEM((2, L), jnp.float32),
        pltpu.VMEM((2, L), jnp.float32),
        pltpu.VMEM((2 * L, PAIR_DIM), jnp.float32),
        pltpu.SemaphoreType.DMA,
        pltpu.SemaphoreType.DMA,
        pltpu.SemaphoreType.DMA,
        pltpu.SemaphoreType.DMA,
        pltpu.SemaphoreType.DMA,
        pltpu.SemaphoreType.DMA,
    ],
  )


@jax.jit
def _impl(sequence_int, dihedral_features, pairing_probs, positional_entropy,
          coupling_matrix, accessibility, conservation, emb_table, pe,
          rel_emb, W_res, b_res, W_pair, b_pair):
    res, relproj = pl.pallas_call(
        _prep_body,
        out_shape=(
            jax.ShapeDtypeStruct((B, L, RES_DIM), jnp.float32),
            jax.ShapeDtypeStruct((NREL, PAIR_DIM), jnp.float32),
        ),
    )(sequence_int.astype(jnp.int32), dihedral_features, positional_entropy,
      accessibility, conservation, emb_table, pe, rel_emb, W_res, b_res,
      W_pair, b_pair)

    pair = _sc_pair()(relproj, W_pair[0:2, :], pairing_probs, coupling_matrix)
    return res, pair


def kernel(sequence_int, mask, dihedral_features, pairing_probs,
           positional_entropy, coupling_matrix, accessibility, conservation,
           emb_table, pe, rel_emb, W_res, b_res, W_pair, b_pair):
    res, pair = _impl(sequence_int, dihedral_features, pairing_probs,
                      positional_entropy, coupling_matrix, accessibility,
                      conservation, emb_table, pe, rel_emb, W_res, b_res,
                      W_pair, b_pair)
    return res, pair, mask
